# baseline (speedup 1.0000x reference)
import functools

import jax
import jax.numpy as jnp
from jax import lax
from jax.experimental import pallas as pl
from jax.experimental.pallas import tpu as pltpu
from jax.experimental.pallas import tpu_sc as plsc

B = 16384
L = 50
D = 64
DW = D // 2
V = 1000000
N_TOK = B * L

NW = 32
BW = B // NW
NG = BW // 128

TR = V // NW
RC = 125
RCH = TR // RC

SENTINEL = 1


def _make_kernel():
    mesh = plsc.VectorSubcoreMesh(core_axis_name="c", subcore_axis_name="s")

    @functools.partial(
        pl.kernel,
        mesh=mesh,
        compiler_params=pltpu.CompilerParams(
            use_tc_tiling_on_sc=False, needs_layout_passes=False),
        out_type=(
            jax.ShapeDtypeStruct((L, DW, 2 * B), jnp.float16),
            jax.ShapeDtypeStruct((NW, 128), jnp.int32),
        ),
        scratch_types=[
            pltpu.HBM((V, DW), jnp.int32),
            pltpu.VMEM((RC, D), jnp.float16),
            pltpu.VMEM((RC, D), jnp.float16),
            pltpu.VMEM((RC, DW), jnp.int32),
            pltpu.VMEM((L, BW), jnp.int32),
            pltpu.VMEM((BW, DW), jnp.int32),
            pltpu.VMEM((BW, DW), jnp.int32),
            pltpu.VMEM((DW, 2 * BW), jnp.float16),
            pltpu.VMEM((NW, 128), jnp.int32),
            pltpu.VMEM((1, 128), jnp.int32),
            pltpu.SemaphoreType.DMA,
            pltpu.SemaphoreType.DMA,
        ],
    )
    def gather_kernel(tok_hbm, tab_hbm, out_hbm, flags_hbm,
                      tabw, f16_a, f16_b, i32_v, idx_v, rows_a, rows_b,
                      pl_v, flag_v, const_v, sem_a, sem_b):
        wid = lax.axis_index("s") * 2 + lax.axis_index("c")

        for k in range(8):
            const_v[0, pl.ds(16 * k, 16)] = jnp.zeros((16,), jnp.int32)
        pltpu.sync_copy(const_v, flags_hbm.at[pl.ds(wid, 1)])

        rbase = wid * TR

        def rconv(buf):
            def conv(i, carry2):
                for u in range(10):
                    row = i * 5 + (u // 2)
                    h = u % 2
                    half = buf[row, pl.ds(32 * h, 32)]
                    i32_v[row, pl.ds(16 * h, 16)] = plsc.bitcast(
                        half, jnp.int32)
                return carry2
            lax.fori_loop(0, RC * 2 // 10, conv, 0)

        pltpu.async_copy(tab_hbm.at[pl.ds(rbase, RC)], f16_a, sem_a)

        def repack_pair(cc, carry):
            c0 = 2 * cc
            pltpu.async_copy(
                tab_hbm.at[pl.ds(rbase + (c0 + 1) * RC, RC)], f16_b, sem_b)
            pltpu.make_async_copy(
                tab_hbm.at[pl.ds(rbase, RC)], f16_a, sem_a).wait()
            rconv(f16_a)
            pltpu.sync_copy(i32_v, tabw.at[pl.ds(rbase + c0 * RC, RC)])

            @pl.when(cc < RCH // 2 - 1)
            def _():
                pltpu.async_copy(
                    tab_hbm.at[pl.ds(rbase + (c0 + 2) * RC, RC)],
                    f16_a, sem_a)

            pltpu.make_async_copy(
                tab_hbm.at[pl.ds(rbase, RC)], f16_b, sem_b).wait()
            rconv(f16_b)
            pltpu.sync_copy(i32_v, tabw.at[pl.ds(rbase + (c0 + 1) * RC, RC)])
            return carry

        lax.fori_loop(0, RCH // 2, repack_pair, 0)

        for k in range(8):
            const_v[0, pl.ds(16 * k, 16)] = jnp.full(
                (16,), SENTINEL, jnp.int32)
        pltpu.sync_copy(const_v, flags_hbm.at[pl.ds(wid, 1)])

        ones16 = jnp.full((16,), 1, jnp.int32)
        zeros16 = jnp.zeros((16,), jnp.int32)

        def spin_cond(n):
            return n != NW * 128

        def spin_body(n):
            pltpu.sync_copy(flags_hbm, flag_v)
            acc = zeros16

            def row_acc(r, a):
                for k in range(8):
                    chunk = flag_v[r, pl.ds(16 * k, 16)]
                    a = a + jnp.where(chunk == SENTINEL, ones16, zeros16)
                return a

            acc = lax.fori_loop(0, NW, row_acc, acc)
            return jnp.sum(acc)

        lax.while_loop(spin_cond, spin_body, jnp.int32(0))

        b0 = wid * BW
        pltpu.sync_copy(tok_hbm.at[:, pl.ds(b0, BW)], idx_v)
        lane = lax.iota(jnp.int32, 16)

        def fire(li, rows_buf, sem):
            for q in range(NG):
                pltpu.async_copy(
                    tabw.at[idx_v.at[li, pl.ds(128 * q, 128)]],
                    rows_buf.at[pl.ds(128 * q, 128)], sem)

        def drain(rows_buf, sem):
            for q in range(NG):
                pltpu.make_async_copy(
                    tabw.at[pl.ds(0, 128)],
                    rows_buf.at[pl.ds(128 * q, 128)], sem).wait()

        def to_planes(li, rows_buf, plane_buf):
            def kloop(k, carry2):
                for g in range(BW // 16):
                    w = plsc.load_gather(
                        rows_buf, [16 * g + lane, k + lane * 0])
                    plane_buf[k, pl.ds(32 * g, 32)] = plsc.bitcast(
                        w, jnp.float16)
                return carry2
            lax.fori_loop(0, DW, kloop, 0)
            pltpu.sync_copy(plane_buf, out_hbm.at[li, :, pl.ds(2 * b0, 2 * BW)])

        fire(0, rows_a, sem_a)

        def gather_pair(ll, carry):
            l0 = 2 * ll
            fire(l0 + 1, rows_b, sem_b)
            drain(rows_a, sem_a)
            to_planes(l0, rows_a, pl_v)

            @pl.when(ll < L // 2 - 1)
            def _():
                fire(l0 + 2, rows_a, sem_a)

            drain(rows_b, sem_b)
            to_planes(l0 + 1, rows_b, pl_v)
            return carry

        lax.fori_loop(0, L // 2, gather_pair, 0)

    return gather_kernel


_gather = _make_kernel()


def kernel(tokens, table):
    out3, _ = _gather(tokens.T, table)
    x = out3.reshape(L, DW, B, 2)
    x = x.transpose(2, 0, 1, 3)
    return x.reshape(B, L, D)[None]


# speedup vs baseline: 1.6300x; 1.6300x over previous
import functools

import jax
import jax.numpy as jnp
from jax import lax
from jax.experimental import pallas as pl
from jax.experimental.pallas import tpu as pltpu
from jax.experimental.pallas import tpu_sc as plsc

B = 16384
L = 50
D = 64
DW = D // 2
V = 1000000
N_TOK = B * L

NW = 32
CH = 100
PER_W = N_TOK // NW
NCHUNK = PER_W // CH

TR = V // NW
RC = 625
RCH = TR // RC

SENTINEL = 1


def _make_kernel():
    mesh = plsc.VectorSubcoreMesh(core_axis_name="c", subcore_axis_name="s")

    @functools.partial(
        pl.kernel,
        mesh=mesh,
        compiler_params=pltpu.CompilerParams(
            use_tc_tiling_on_sc=False, needs_layout_passes=False),
        out_type=(
            jax.ShapeDtypeStruct((1, B, L, D), jnp.float16),
            jax.ShapeDtypeStruct((NW, 128), jnp.int32),
        ),
        scratch_types=[
            pltpu.HBM((V, DW), jnp.int32),
            pltpu.VMEM((RC, D), jnp.float16),
            pltpu.VMEM((RC, D), jnp.float16),
            pltpu.VMEM((RC, DW), jnp.int32),
            pltpu.VMEM((NCHUNK, CH), jnp.int32),
            pltpu.VMEM((CH, DW), jnp.int32),
            pltpu.VMEM((CH, DW), jnp.int32),
            pltpu.VMEM((2, L, D), jnp.float16),
            pltpu.VMEM((NW, 128), jnp.int32),
            pltpu.VMEM((1, 128), jnp.int32),
            pltpu.SemaphoreType.DMA,
            pltpu.SemaphoreType.DMA,
        ],
    )
    def gather_kernel(tok_hbm, tab_hbm, out_hbm, flags_hbm,
                      tabw, f16_a, f16_b, i32_v, idx_v, rows_a, rows_b,
                      outb_v, flag_v, const_v, sem_a, sem_b):
        wid = lax.axis_index("s") * 2 + lax.axis_index("c")

        for k in range(8):
            const_v[0, pl.ds(16 * k, 16)] = jnp.zeros((16,), jnp.int32)
        pltpu.sync_copy(const_v, flags_hbm.at[pl.ds(wid, 1)])

        rbase = wid * TR

        def rconv(buf):
            def conv(i, carry2):
                for u in range(10):
                    row = i * 5 + (u // 2)
                    h = u % 2
                    half = buf[row, pl.ds(32 * h, 32)]
                    i32_v[row, pl.ds(16 * h, 16)] = plsc.bitcast(
                        half, jnp.int32)
                return carry2
            lax.fori_loop(0, RC * 2 // 10, conv, 0)

        pltpu.async_copy(tab_hbm.at[pl.ds(rbase, RC)], f16_a, sem_a)

        def repack_pair(cc, carry):
            c0 = 2 * cc
            pltpu.async_copy(
                tab_hbm.at[pl.ds(rbase + (c0 + 1) * RC, RC)], f16_b, sem_b)
            pltpu.make_async_copy(
                tab_hbm.at[pl.ds(rbase, RC)], f16_a, sem_a).wait()
            rconv(f16_a)
            pltpu.sync_copy(i32_v, tabw.at[pl.ds(rbase + c0 * RC, RC)])

            @pl.when(cc < RCH // 2 - 1)
            def _():
                pltpu.async_copy(
                    tab_hbm.at[pl.ds(rbase + (c0 + 2) * RC, RC)],
                    f16_a, sem_a)

            pltpu.make_async_copy(
                tab_hbm.at[pl.ds(rbase, RC)], f16_b, sem_b).wait()
            rconv(f16_b)
            pltpu.sync_copy(i32_v, tabw.at[pl.ds(rbase + (c0 + 1) * RC, RC)])
            return carry

        lax.fori_loop(0, RCH // 2, repack_pair, 0)

        for k in range(8):
            const_v[0, pl.ds(16 * k, 16)] = jnp.full(
                (16,), SENTINEL, jnp.int32)
        pltpu.sync_copy(const_v, flags_hbm.at[pl.ds(wid, 1)])

        ones16 = jnp.full((16,), 1, jnp.int32)
        zeros16 = jnp.zeros((16,), jnp.int32)

        def spin_cond(n):
            return n != NW * 128

        def spin_body(n):
            pltpu.sync_copy(flags_hbm, flag_v)
            acc = zeros16

            def row_acc(r, a):
                for k in range(8):
                    chunk = flag_v[r, pl.ds(16 * k, 16)]
                    a = a + jnp.where(chunk == SENTINEL, ones16, zeros16)
                return a

            acc = lax.fori_loop(0, NW, row_acc, acc)
            return jnp.sum(acc)

        lax.while_loop(spin_cond, spin_body, jnp.int32(0))

        pltpu.sync_copy(tok_hbm.at[pl.ds(wid * NCHUNK, NCHUNK)], idx_v)
        bbase = wid * (B // NW)

        def gconv(buf, j):
            def conv(i, carry2):
                for u in range(10):
                    tok_i = i * 5 + (u // 2)
                    h = u % 2
                    w16 = plsc.bitcast(
                        buf[tok_i, pl.ds(16 * h, 16)], jnp.float16)
                    outb_v[tok_i // L, tok_i % L, pl.ds(32 * h, 32)] = w16
                return carry2
            lax.fori_loop(0, CH * 2 // 10, conv, 0)
            pltpu.sync_copy(outb_v, out_hbm.at[0, pl.ds(bbase + 2 * j, 2)])

        pltpu.async_copy(tabw.at[idx_v.at[0]], rows_a, sem_a)

        def gather_pair(jj, carry):
            j0 = 2 * jj
            pltpu.async_copy(tabw.at[idx_v.at[j0 + 1]], rows_b, sem_b)
            pltpu.make_async_copy(
                tabw.at[pl.ds(0, CH)], rows_a, sem_a).wait()
            gconv(rows_a, j0)

            @pl.when(jj < NCHUNK // 2 - 1)
            def _():
                pltpu.async_copy(tabw.at[idx_v.at[j0 + 2]], rows_a, sem_a)

            pltpu.make_async_copy(
                tabw.at[pl.ds(0, CH)], rows_b, sem_b).wait()
            gconv(rows_b, j0 + 1)
            return carry

        lax.fori_loop(0, NCHUNK // 2, gather_pair, 0)

    return gather_kernel


_gather = _make_kernel()


def kernel(tokens, table):
    tok = tokens.reshape(NW * NCHUNK, CH)
    out, _ = _gather(tok, table)
    return out
